# Initial kernel scaffold; baseline (speedup 1.0000x reference)
import functools

import jax
import jax.numpy as jnp
from jax import lax
from jax.experimental import pallas as pl
from jax.experimental.pallas import tpu as pltpu
from jax.experimental.pallas import tpu_sc as plsc

_NC = 2
_NS = 16
_CH = 80


def _make_sc_agg(n, d, e, with_cnt):
    rows_per_tile = n // _NS
    ept = e // (_NC * _NS)
    n_chunks = ept // _CH
    assert ept % _CH == 0 and n % _NS == 0 and e % (_NC * _NS) == 0

    out_type = [jax.ShapeDtypeStruct((_NC, n, d), jnp.float32)]
    scratch = [
        pltpu.VMEM((_CH,), jnp.int32),
        pltpu.VMEM((_CH,), jnp.int32),
        pltpu.VMEM((_CH, d), jnp.float32),
        pltpu.VMEM_SHARED((n, d), jnp.float32),
    ]
    if with_cnt:
        out_type.append(jax.ShapeDtypeStruct((_NC, n, 16), jnp.float32))
        scratch += [
            pltpu.VMEM((_CH, 16), jnp.float32),
            pltpu.VMEM_SHARED((n, 16), jnp.float32),
        ]

    def body(*refs):
        if with_cnt:
            (h_hbm, src_hbm, dst_hbm, zrow_hbm, zcnt_hbm, ones_hbm,
             p_hbm, cnt_hbm, src_v, dst_v, rows_v, agg_sh, ones_v, cnt_sh) = refs
        else:
            (h_hbm, src_hbm, dst_hbm, zrow_hbm,
             p_hbm, src_v, dst_v, rows_v, agg_sh) = refs
        c = lax.axis_index("c")
        s = lax.axis_index("s")
        r0 = s * rows_per_tile
        pltpu.sync_copy(zrow_hbm, agg_sh.at[pl.ds(r0, rows_per_tile)])
        if with_cnt:
            pltpu.sync_copy(zcnt_hbm, cnt_sh.at[pl.ds(r0, rows_per_tile)])
            pltpu.sync_copy(ones_hbm, ones_v)
        plsc.subcore_barrier()

        base = (c * _NS + s) * ept

        def step(i, carry):
            off = base + i * _CH
            pltpu.sync_copy(src_hbm.at[pl.ds(off, _CH)], src_v)
            pltpu.sync_copy(dst_hbm.at[pl.ds(off, _CH)], dst_v)
            pltpu.sync_copy(h_hbm.at[src_v], rows_v)
            pltpu.sync_copy(rows_v, agg_sh.at[dst_v], add=True)
            if with_cnt:
                pltpu.sync_copy(ones_v, cnt_sh.at[dst_v], add=True)
            return carry

        lax.fori_loop(0, n_chunks, step, 0)
        plsc.subcore_barrier()
        pltpu.sync_copy(agg_sh.at[pl.ds(r0, rows_per_tile)],
                        p_hbm.at[c, pl.ds(r0, rows_per_tile)])
        if with_cnt:
            pltpu.sync_copy(cnt_sh.at[pl.ds(r0, rows_per_tile)],
                            cnt_hbm.at[c, pl.ds(r0, rows_per_tile)])

    mesh = plsc.VectorSubcoreMesh(core_axis_name="c", subcore_axis_name="s")
    return pl.kernel(body, out_type=out_type, mesh=mesh, scratch_types=scratch)


def _layer_math(p, cnt2, h, wl, bl, wr, g, b):
    agg = p[0] + p[1]
    cnt = cnt2[0, :, 0] + cnt2[1, :, 0]
    mean = agg / jnp.maximum(cnt, 1.0)[:, None]
    out = lax.dot_general(mean, wl, (((1,), (1,)), ((), ())),
                          preferred_element_type=jnp.float32) + bl[None, :]
    out = out + lax.dot_general(h, wr, (((1,), (1,)), ((), ())),
                                preferred_element_type=jnp.float32)
    mu = jnp.mean(out, axis=-1, keepdims=True)
    var = jnp.mean((out - mu) ** 2, axis=-1, keepdims=True)
    y = (out - mu) * lax.rsqrt(var + 1e-5) * g[None, :] + b[None, :]
    return jnp.maximum(y, 0.0)


def _tc_layer_body(p_ref, cnt_ref, h_ref, wl_ref, bl_ref, wr_ref, g_ref, b_ref,
                   o_ref):
    o_ref[...] = _layer_math(p_ref[...], cnt_ref[...], h_ref[...], wl_ref[...],
                             bl_ref[...], wr_ref[...], g_ref[...], b_ref[...])


def _tc_final_body(p_ref, cnt_ref, h_ref, wl_ref, bl_ref, wr_ref, g_ref, b_ref,
                   wo_ref, bo_ref, o_ref):
    hr = _layer_math(p_ref[...], cnt_ref[...], h_ref[...], wl_ref[...],
                     bl_ref[...], wr_ref[...], g_ref[...], b_ref[...])
    logits = lax.dot_general(hr, wo_ref[...], (((1,), (1,)), ((), ())),
                             preferred_element_type=jnp.float32) + bo_ref[...][None, :]
    m = jnp.max(logits, axis=-1, keepdims=True)
    lse = jnp.log(jnp.sum(jnp.exp(logits - m), axis=-1, keepdims=True)) + m
    o_ref[...] = logits - lse


_BLK = 400


def _make_tc_layer(n, d):
    grid = (n // _BLK,)
    in_specs = [
        pl.BlockSpec((_NC, _BLK, d), lambda i: (0, i, 0)),
        pl.BlockSpec((_NC, _BLK, 16), lambda i: (0, i, 0)),
        pl.BlockSpec((_BLK, d), lambda i: (i, 0)),
        pl.BlockSpec((d, d), lambda i: (0, 0)),
        pl.BlockSpec((d,), lambda i: (0,)),
        pl.BlockSpec((d, d), lambda i: (0, 0)),
        pl.BlockSpec((d,), lambda i: (0,)),
        pl.BlockSpec((d,), lambda i: (0,)),
    ]
    return pl.pallas_call(
        _tc_layer_body,
        grid=grid,
        in_specs=in_specs,
        out_specs=pl.BlockSpec((_BLK, d), lambda i: (i, 0)),
        out_shape=jax.ShapeDtypeStruct((n, d), jnp.float32),
    )


def _make_tc_final(n, d, c_out):
    grid = (n // _BLK,)
    in_specs = [
        pl.BlockSpec((_NC, _BLK, d), lambda i: (0, i, 0)),
        pl.BlockSpec((_NC, _BLK, 16), lambda i: (0, i, 0)),
        pl.BlockSpec((_BLK, d), lambda i: (i, 0)),
        pl.BlockSpec((d, d), lambda i: (0, 0)),
        pl.BlockSpec((d,), lambda i: (0,)),
        pl.BlockSpec((d, d), lambda i: (0, 0)),
        pl.BlockSpec((d,), lambda i: (0,)),
        pl.BlockSpec((d,), lambda i: (0,)),
        pl.BlockSpec((c_out, d), lambda i: (0, 0)),
        pl.BlockSpec((c_out,), lambda i: (0,)),
    ]
    return pl.pallas_call(
        _tc_final_body,
        grid=grid,
        in_specs=in_specs,
        out_specs=pl.BlockSpec((_BLK, c_out), lambda i: (i, 0)),
        out_shape=jax.ShapeDtypeStruct((n, c_out), jnp.float32),
    )


def kernel(x, edge_index, Wl0, bl0, Wr0, g0, b0, Wl1, bl1, Wr1, g1, b1,
           Wl2, bl2, Wr2, g2, b2, Wout, bout):
    n, d = x.shape
    e = edge_index.shape[1]
    c_out = Wout.shape[0]
    dst = edge_index[0]
    src = edge_index[1]
    zrow = jnp.zeros((n // _NS, d), jnp.float32)
    zcnt = jnp.zeros((n // _NS, 16), jnp.float32)
    ones = jnp.ones((_CH, 16), jnp.float32)

    sc_agg_cnt = _make_sc_agg(n, d, e, with_cnt=True)
    sc_agg = _make_sc_agg(n, d, e, with_cnt=False)
    tc_layer = _make_tc_layer(n, d)
    tc_final = _make_tc_final(n, d, c_out)

    p0, cnt2 = sc_agg_cnt(x, src, dst, zrow, zcnt, ones)
    h1 = tc_layer(p0, cnt2, x, Wl0, bl0, Wr0, g0, b0)
    p1 = sc_agg(h1, src, dst, zrow)
    h2 = tc_layer(p1, cnt2, h1, Wl1, bl1, Wr1, g1, b1)
    p2 = sc_agg(h2, src, dst, zrow)
    return tc_final(p2, cnt2, h2, Wl2, bl2, Wr2, g2, b2, Wout, bout)


# speedup vs baseline: 4.2153x; 4.2153x over previous
import jax
import jax.numpy as jnp
from jax import lax
from jax.experimental import pallas as pl
from jax.experimental.pallas import tpu as pltpu
from jax.experimental.pallas import tpu_sc as plsc

_NC = 2
_NS = 16
_CH = 80


def _make_sc_agg(n_pad, d, e):
    rows_per_tile = n_pad // _NS
    ept = e // (_NC * _NS)
    n_chunks = ept // _CH
    assert ept % _CH == 0 and rows_per_tile % 8 == 0 and e % (_NC * _NS) == 0

    def body(h_hbm, src_hbm, dst_hbm, zrow_hbm, p_hbm,
             src_v, dst_v, rows_v, agg_sh):
        c = lax.axis_index("c")
        s = lax.axis_index("s")
        r0 = s * rows_per_tile
        pltpu.sync_copy(zrow_hbm, agg_sh.at[pl.ds(r0, rows_per_tile)])
        plsc.subcore_barrier()

        base = (c * _NS + s) * ept

        def step(i, carry):
            off = base + i * _CH
            pltpu.sync_copy(src_hbm.at[pl.ds(off, _CH)], src_v)
            pltpu.sync_copy(dst_hbm.at[pl.ds(off, _CH)], dst_v)
            pltpu.sync_copy(h_hbm.at[src_v], rows_v)
            pltpu.sync_copy(rows_v, agg_sh.at[dst_v], add=True)
            return carry

        lax.fori_loop(0, n_chunks, step, 0)
        plsc.subcore_barrier()
        pltpu.sync_copy(agg_sh.at[pl.ds(r0, rows_per_tile)],
                        p_hbm.at[c, pl.ds(r0, rows_per_tile)])

    mesh = plsc.VectorSubcoreMesh(core_axis_name="c", subcore_axis_name="s")
    return pl.kernel(
        body,
        out_type=jax.ShapeDtypeStruct((_NC, n_pad, d), jnp.float32),
        mesh=mesh,
        scratch_types=[
            pltpu.VMEM((_CH,), jnp.int32),
            pltpu.VMEM((_CH,), jnp.int32),
            pltpu.VMEM((_CH, d), jnp.float32),
            pltpu.VMEM_SHARED((n_pad, d), jnp.float32),
        ],
    )


def _make_sc_cnt(n_pad, d, e):
    rows_per_tile = n_pad // _NS
    ept = e // (_NC * _NS)
    n_chunks = ept // _CH

    def body(dst_hbm, zrow_hbm, ones_hbm, cnt_hbm, dst_v, ones_v, cnt_sh):
        c = lax.axis_index("c")
        s = lax.axis_index("s")
        r0 = s * rows_per_tile
        pltpu.sync_copy(zrow_hbm, cnt_sh.at[pl.ds(r0, rows_per_tile)])
        pltpu.sync_copy(ones_hbm, ones_v)
        plsc.subcore_barrier()

        base = (c * _NS + s) * ept

        def step(i, carry):
            off = base + i * _CH
            pltpu.sync_copy(dst_hbm.at[pl.ds(off, _CH)], dst_v)
            pltpu.sync_copy(ones_v, cnt_sh.at[dst_v], add=True)
            return carry

        lax.fori_loop(0, n_chunks, step, 0)
        plsc.subcore_barrier()
        pltpu.sync_copy(cnt_sh.at[pl.ds(r0, rows_per_tile)],
                        cnt_hbm.at[c, pl.ds(r0, rows_per_tile)])

    mesh = plsc.VectorSubcoreMesh(core_axis_name="c", subcore_axis_name="s")
    return pl.kernel(
        body,
        out_type=jax.ShapeDtypeStruct((_NC, n_pad, d), jnp.float32),
        mesh=mesh,
        scratch_types=[
            pltpu.VMEM((_CH,), jnp.int32),
            pltpu.VMEM((_CH, d), jnp.float32),
            pltpu.VMEM_SHARED((n_pad, d), jnp.float32),
        ],
    )


def _layer_math(p, cnth, h, wl, bl, wr, g, b):
    agg = p[0] + p[1]
    cnt = cnth[0, :, 0] + cnth[1, :, 0]
    mean = agg / jnp.maximum(cnt, 1.0)[:, None]
    out = lax.dot_general(mean, wl, (((1,), (1,)), ((), ())),
                          preferred_element_type=jnp.float32) + bl[None, :]
    out = out + lax.dot_general(h, wr, (((1,), (1,)), ((), ())),
                                preferred_element_type=jnp.float32)
    mu = jnp.mean(out, axis=-1, keepdims=True)
    var = jnp.mean((out - mu) ** 2, axis=-1, keepdims=True)
    y = (out - mu) * lax.rsqrt(var + 1e-5) * g[None, :] + b[None, :]
    return jnp.maximum(y, 0.0)


def _tc_layer_body(p_ref, cnt_ref, h_ref, wl_ref, bl_ref, wr_ref, g_ref, b_ref,
                   o_ref):
    o_ref[...] = _layer_math(p_ref[...], cnt_ref[...], h_ref[...], wl_ref[...],
                             bl_ref[...], wr_ref[...], g_ref[...], b_ref[...])


def _tc_final_body(p_ref, cnt_ref, h_ref, wl_ref, bl_ref, wr_ref, g_ref, b_ref,
                   wo_ref, bo_ref, o_ref):
    hr = _layer_math(p_ref[...], cnt_ref[...], h_ref[...], wl_ref[...],
                     bl_ref[...], wr_ref[...], g_ref[...], b_ref[...])
    logits = lax.dot_general(hr, wo_ref[...], (((1,), (1,)), ((), ())),
                             preferred_element_type=jnp.float32) + bo_ref[...][None, :]
    m = jnp.max(logits, axis=-1, keepdims=True)
    lse = jnp.log(jnp.sum(jnp.exp(logits - m), axis=-1, keepdims=True)) + m
    o_ref[...] = logits - lse


_BLK = 400


def _make_tc_layer(n, d):
    grid = (n // _BLK,)
    in_specs = [
        pl.BlockSpec((_NC, _BLK, d), lambda i: (0, i, 0)),
        pl.BlockSpec((_NC, _BLK, d), lambda i: (0, i, 0)),
        pl.BlockSpec((_BLK, d), lambda i: (i, 0)),
        pl.BlockSpec((d, d), lambda i: (0, 0)),
        pl.BlockSpec((d,), lambda i: (0,)),
        pl.BlockSpec((d, d), lambda i: (0, 0)),
        pl.BlockSpec((d,), lambda i: (0,)),
        pl.BlockSpec((d,), lambda i: (0,)),
    ]
    return pl.pallas_call(
        _tc_layer_body,
        grid=grid,
        in_specs=in_specs,
        out_specs=pl.BlockSpec((_BLK, d), lambda i: (i, 0)),
        out_shape=jax.ShapeDtypeStruct((n, d), jnp.float32),
    )


def _make_tc_final(n, d, c_out):
    grid = (n // _BLK,)
    in_specs = [
        pl.BlockSpec((_NC, _BLK, d), lambda i: (0, i, 0)),
        pl.BlockSpec((_NC, _BLK, d), lambda i: (0, i, 0)),
        pl.BlockSpec((_BLK, d), lambda i: (i, 0)),
        pl.BlockSpec((d, d), lambda i: (0, 0)),
        pl.BlockSpec((d,), lambda i: (0,)),
        pl.BlockSpec((d, d), lambda i: (0, 0)),
        pl.BlockSpec((d,), lambda i: (0,)),
        pl.BlockSpec((d,), lambda i: (0,)),
        pl.BlockSpec((c_out, d), lambda i: (0, 0)),
        pl.BlockSpec((c_out,), lambda i: (0,)),
    ]
    return pl.pallas_call(
        _tc_final_body,
        grid=grid,
        in_specs=in_specs,
        out_specs=pl.BlockSpec((_BLK, c_out), lambda i: (i, 0)),
        out_shape=jax.ShapeDtypeStruct((n, c_out), jnp.float32),
    )


def kernel(x, edge_index, Wl0, bl0, Wr0, g0, b0, Wl1, bl1, Wr1, g1, b1,
           Wl2, bl2, Wr2, g2, b2, Wout, bout):
    n, d = x.shape
    e = edge_index.shape[1]
    c_out = Wout.shape[0]
    n_pad = ((n + _NS * 8 - 1) // (_NS * 8)) * (_NS * 8)
    dst = edge_index[0]
    src = edge_index[1]
    zrow = jnp.zeros((n_pad // _NS, d), jnp.float32)
    ones = jnp.ones((_CH, d), jnp.float32)

    sc_agg = _make_sc_agg(n_pad, d, e)
    sc_cnt = _make_sc_cnt(n_pad, d, e)
    tc_layer = _make_tc_layer(n, d)
    tc_final = _make_tc_final(n, d, c_out)

    cnth = sc_cnt(dst, zrow, ones)
    p0 = sc_agg(x, src, dst, zrow)
    h1 = tc_layer(p0, cnth, x, Wl0, bl0, Wr0, g0, b0)
    p1 = sc_agg(h1, src, dst, zrow)
    h2 = tc_layer(p1, cnth, h1, Wl1, bl1, Wr1, g1, b1)
    p2 = sc_agg(h2, src, dst, zrow)
    return tc_final(p2, cnth, h2, Wl2, bl2, Wr2, g2, b2, Wout, bout)
